# Initial kernel scaffold; baseline (speedup 1.0000x reference)
import jax
import jax.numpy as jnp
from jax.experimental import pallas as pl


def kernel(x, signs, perm):
    raise NotImplementedError("write your pallas kernel here")


# speedup vs baseline: 2.3096x; 2.3096x over previous
import functools

import jax
import jax.numpy as jnp
from jax import lax
from jax.experimental import pallas as pl
from jax.experimental.pallas import tpu as pltpu
from jax.experimental.pallas import tpu_sc as plsc

B = 4
S = 8192
D = 1024
L = 16
NC = 2
NS = 16
NW = NC * NS
R = B * S
RPW = R // NW
CH = 32
NCH = RPW // CH


def _body(xf, sgn, perm, out, perm_v, sgn_v, s_v, buf, sem):
    wid = lax.axis_index("s") * NC + lax.axis_index("c")
    wbase = wid * RPW
    bofs = (wbase // S) * S
    ibase = wbase - bofs

    pltpu.sync_copy(perm.at[pl.ds(ibase, RPW)], perm_v)
    pltpu.sync_copy(sgn, sgn_v)

    vofs = jnp.full((L,), bofs, jnp.int32)

    def prep(k, _):
        idx16 = perm_v[pl.ds(k * L, L)]
        s_v[pl.ds(k * L, L)] = plsc.load_gather(sgn_v, [idx16])
        perm_v[pl.ds(k * L, L)] = idx16 + vofs
        return 0

    lax.fori_loop(0, RPW // L, prep, 0, unroll=2)

    def do_chunk(c, _):
        pltpu.async_copy(xf.at[perm_v.at[pl.ds(c * CH, CH)]], buf, sem).wait()

        def row(r, _):
            svec = plsc.load_gather(s_v, [jnp.full((L,), c * CH + r, jnp.int32)])
            for j in range(D // L):
                sl = pl.ds(j * L, L)
                buf[r, sl] = buf[r, sl] * svec
            return 0

        lax.fori_loop(0, CH, row, 0)
        pltpu.sync_copy(buf, out.at[pl.ds(wbase + c * CH, CH)])
        return 0

    lax.fori_loop(0, NCH, do_chunk, 0)


@jax.jit
def kernel(x, signs, perm):
    xf = x.reshape(R, D)
    sgn = signs.reshape(S)
    mesh = plsc.VectorSubcoreMesh(core_axis_name="c", subcore_axis_name="s",
                                  num_cores=NC, num_subcores=NS)
    out = pl.kernel(
        _body,
        out_type=jax.ShapeDtypeStruct((R, D), jnp.float32),
        mesh=mesh,
        scratch_types=[
            pltpu.VMEM((RPW,), jnp.int32),
            pltpu.VMEM((S,), jnp.float32),
            pltpu.VMEM((RPW,), jnp.float32),
            pltpu.VMEM((CH, D), jnp.float32),
            pltpu.SemaphoreType.DMA,
        ],
        compiler_params=pltpu.CompilerParams(needs_layout_passes=False),
    )(xf, sgn, perm)
    return out.reshape(B, S, D)


# speedup vs baseline: 4.1083x; 1.7788x over previous
import jax
import jax.numpy as jnp
from jax import lax
from jax.experimental import pallas as pl
from jax.experimental.pallas import tpu as pltpu
from jax.experimental.pallas import tpu_sc as plsc

B = 4
S = 8192
D = 1024
L = 16
NC = 2
NS = 16
NW = NC * NS
R = B * S
RPW = R // NW
CH = 16
NCH = RPW // CH
NBUF = 4


def _body(xf, sgn, perm, out, perm_v, sgn_v, s_v, bufs, gsems, ssems):
    wid = lax.axis_index("s") * NC + lax.axis_index("c")
    wbase = wid * RPW
    bofs = (wbase // S) * S
    ibase = wbase - bofs

    pltpu.sync_copy(perm.at[pl.ds(ibase, RPW)], perm_v)
    pltpu.sync_copy(sgn, sgn_v)

    vofs = jnp.full((L,), bofs, jnp.int32)

    def prep(k, _):
        idx16 = perm_v[pl.ds(k * L, L)]
        s_v[pl.ds(k * L, L)] = plsc.load_gather(sgn_v, [idx16])
        perm_v[pl.ds(k * L, L)] = idx16 + vofs
        return 0

    lax.fori_loop(0, RPW // L, prep, 0, unroll=2)

    def start_gather(c, b):
        pltpu.async_copy(xf.at[perm_v.at[pl.ds(c * CH, CH)]], bufs[b], gsems[b])

    def wait_sem(sem, b):
        pltpu.make_async_copy(xf.at[pl.ds(0, CH)], bufs[b], sem).wait()

    start_gather(0, 0)
    start_gather(1, 1)

    def group(g, _):
        c0 = g * NBUF
        for b in range(NBUF):
            c = c0 + b
            bn = (b + 2) % NBUF

            @pl.when(c >= 2)
            def _():
                wait_sem(ssems[bn], bn)

            @pl.when(c + 2 < NCH)
            def _():
                start_gather(c + 2, bn)

            wait_sem(gsems[b], b)

            def row(r, _):
                svec = plsc.load_gather(
                    s_v, [jnp.full((L,), c * CH + r, jnp.int32)])
                for j in range(D // L):
                    sl = pl.ds(j * L, L)
                    bufs[b][r, sl] = bufs[b][r, sl] * svec
                return 0

            lax.fori_loop(0, CH, row, 0)
            pltpu.async_copy(bufs[b], out.at[pl.ds(wbase + c * CH, CH)],
                             ssems[b])
        return 0

    lax.fori_loop(0, NCH // NBUF, group, 0)

    for c in (NCH - 2, NCH - 1):
        b = c % NBUF
        wait_sem(ssems[b], b)


@jax.jit
def kernel(x, signs, perm):
    xf = x.reshape(R, D)
    sgn = signs.reshape(S)
    mesh = plsc.VectorSubcoreMesh(core_axis_name="c", subcore_axis_name="s",
                                  num_cores=NC, num_subcores=NS)
    out = pl.kernel(
        _body,
        out_type=jax.ShapeDtypeStruct((R, D), jnp.float32),
        mesh=mesh,
        scratch_types=[
            pltpu.VMEM((RPW,), jnp.int32),
            pltpu.VMEM((S,), jnp.float32),
            pltpu.VMEM((RPW,), jnp.float32),
            [pltpu.VMEM((CH, D), jnp.float32) for _ in range(NBUF)],
            [pltpu.SemaphoreType.DMA for _ in range(NBUF)],
            [pltpu.SemaphoreType.DMA for _ in range(NBUF)],
        ],
        compiler_params=pltpu.CompilerParams(needs_layout_passes=False),
    )(xf, sgn, perm)
    return out.reshape(B, S, D)
